# Initial kernel scaffold; baseline (speedup 1.0000x reference)
import jax
import jax.numpy as jnp
from jax.experimental import pallas as pl


def kernel(x, edge_index, W_gcn, b_gcn, W_gat, b_gat, a_src, a_dst, W1, b1, W2, b2, W_fc, b_fc):
    raise NotImplementedError("write your pallas kernel here")


# speedup vs baseline: 18.0432x; 18.0432x over previous
import dataclasses
import functools

import jax
import jax.numpy as jnp
from jax import lax
from jax.experimental import pallas as pl
from jax.experimental.pallas import tpu as pltpu
from jax.experimental.pallas import tpu_sc as plsc

N = 10000
E = 320000
H = 128

NC = 2
NS = 16
L = 16
NW = NC * NS
EPW = E // NW
CH = 80
NCH = EPW // CH
NP = 10240

_mesh = plsc.VectorSubcoreMesh(core_axis_name="c", subcore_axis_name="s")

_cp = pltpu.CompilerParams()
if "needs_layout_passes" in pltpu.CompilerParams.__dataclass_fields__:
    _cp = dataclasses.replace(_cp, needs_layout_passes=False)


def _fill_zeros_1d(stage):
    @pl.loop(0, 640, step=L)
    def _(j):
        stage[pl.ds(j, L)] = jnp.zeros((L,), jnp.float32)


def _fill_zeros_2d(stage):
    @pl.loop(0, CH)
    def _(i):
        row = stage.at[i]
        for kk in range(H // L):
            row[pl.ds(kk * L, L)] = jnp.zeros((L,), jnp.float32)


def _tile_zero_1d(stage, sh_ref, sid):
    pltpu.sync_copy(stage, sh_ref.at[pl.ds(sid * 640, 640)])


def _tile_copyout_1d(sh_ref, out_hbm_core, stage, sid):
    pltpu.sync_copy(sh_ref.at[pl.ds(sid * 640, 640)], stage)
    pltpu.sync_copy(stage, out_hbm_core.at[pl.ds(sid * 640, 640)])


def _tile_zero_2d(stage, sh_ref, sid):
    @pl.when(sid < NS - 1)
    def _():
        @pl.loop(0, 8)
        def _(t):
            pltpu.sync_copy(stage, sh_ref.at[pl.ds(sid * 640 + t * CH, CH)])

    @pl.when(sid == NS - 1)
    def _():
        @pl.loop(0, 5)
        def _(t):
            pltpu.sync_copy(stage, sh_ref.at[pl.ds(9600 + t * CH, CH)])


def _tile_copyout_2d(sh_ref, out_hbm_core, stage, sid):
    @pl.when(sid < NS - 1)
    def _():
        @pl.loop(0, 8)
        def _(t):
            pltpu.sync_copy(sh_ref.at[pl.ds(sid * 640 + t * CH, CH)], stage)
            pltpu.sync_copy(stage, out_hbm_core.at[pl.ds(sid * 640 + t * CH,
                                                         CH)])

    @pl.when(sid == NS - 1)
    def _():
        @pl.loop(0, 5)
        def _(t):
            pltpu.sync_copy(sh_ref.at[pl.ds(9600 + t * CH, CH)], stage)
            pltpu.sync_copy(stage, out_hbm_core.at[pl.ds(9600 + t * CH, CH)])


def _sc_degree(dst_r):
    @functools.partial(
        pl.kernel,
        out_type=jax.ShapeDtypeStruct((NC, NP), jnp.float32),
        mesh=_mesh,
        scratch_types=[
            pltpu.VMEM((NCH, CH), jnp.int32),
            pltpu.VMEM((CH,), jnp.float32),
            pltpu.VMEM((640,), jnp.float32),
            pltpu.VMEM_SHARED((NP,), jnp.float32),
        ],
    )
    def k(dst_hbm, out_hbm, idx_v, ones_v, stage1, deg_sh):
        cid = lax.axis_index("c")
        sid = lax.axis_index("s")
        wid = cid * NS + sid
        pltpu.sync_copy(dst_hbm.at[wid], idx_v)

        @pl.loop(0, CH, step=L)
        def _(j):
            ones_v[pl.ds(j, L)] = jnp.full((L,), 1.0, jnp.float32)

        _fill_zeros_1d(stage1)
        _tile_zero_1d(stage1, deg_sh, sid)
        plsc.subcore_barrier()

        @pl.loop(0, NCH)
        def _(g):
            pltpu.sync_copy(ones_v, deg_sh.at[idx_v.at[g]], add=True)

        plsc.subcore_barrier()
        _tile_copyout_1d(deg_sh, out_hbm.at[cid], stage1, sid)

    return k(dst_r)


def _sc_segsum(src_r, dst_r, table):
    @functools.partial(
        pl.kernel,
        out_type=jax.ShapeDtypeStruct((NC, N, H), jnp.float32),
        mesh=_mesh,
        scratch_types=[
            pltpu.VMEM((NCH, CH), jnp.int32),
            pltpu.VMEM((NCH, CH), jnp.int32),
            pltpu.VMEM((CH, H), jnp.float32),
            pltpu.VMEM_SHARED((N, H), jnp.float32),
        ],
    )
    def k(src_hbm, dst_hbm, tab_hbm, out_hbm,
          isrc_v, idst_v, rows_v, acc_sh):
        cid = lax.axis_index("c")
        sid = lax.axis_index("s")
        wid = cid * NS + sid
        pltpu.sync_copy(src_hbm.at[wid], isrc_v)
        pltpu.sync_copy(dst_hbm.at[wid], idst_v)
        _fill_zeros_2d(rows_v)
        _tile_zero_2d(rows_v, acc_sh, sid)
        plsc.subcore_barrier()

        @pl.loop(0, NCH)
        def _(g):
            pltpu.sync_copy(tab_hbm.at[isrc_v.at[g]], rows_v)
            pltpu.sync_copy(rows_v, acc_sh.at[idst_v.at[g]], add=True)

        plsc.subcore_barrier()
        _tile_copyout_2d(acc_sh, out_hbm.at[cid], rows_v, sid)

    return k(src_r, dst_r, table)


def _sc_gat_logits(src_r, dst_r, s_n, d_n, mp_n):
    @functools.partial(
        pl.kernel,
        out_type=(jax.ShapeDtypeStruct((NC, NP), jnp.float32),
                  jax.ShapeDtypeStruct((NW, NCH, 128), jnp.float32)),
        mesh=_mesh,
        compiler_params=_cp,
        scratch_types=[
            pltpu.VMEM((NCH, CH), jnp.int32),
            pltpu.VMEM((NCH, CH), jnp.int32),
            pltpu.VMEM((NP,), jnp.float32),
            pltpu.VMEM((NP,), jnp.float32),
            pltpu.VMEM((NP,), jnp.float32),
            pltpu.VMEM((NCH, 128), jnp.float32),
            pltpu.VMEM((640,), jnp.float32),
            pltpu.VMEM_SHARED((NP,), jnp.float32),
        ],
    )
    def k(src_hbm, dst_hbm, s_hbm, d_hbm, mp_hbm,
          den_hbm, ex_hbm,
          isrc_v, idst_v, s_v, d_v, mp_v, ex_v, stage1, den_sh):
        cid = lax.axis_index("c")
        sid = lax.axis_index("s")
        wid = cid * NS + sid
        pltpu.sync_copy(src_hbm.at[wid], isrc_v)
        pltpu.sync_copy(dst_hbm.at[wid], idst_v)
        pltpu.sync_copy(s_hbm, s_v)
        pltpu.sync_copy(d_hbm, d_v)
        pltpu.sync_copy(mp_hbm, mp_v)
        _fill_zeros_1d(stage1)
        _tile_zero_1d(stage1, den_sh, sid)
        plsc.subcore_barrier()

        @pl.loop(0, NCH)
        def _(g):
            isrc_row = isrc_v.at[g]
            idst_row = idst_v.at[g]
            ex_row = ex_v.at[g]

            @pl.loop(0, CH, step=L)
            def _(j):
                isrc = isrc_row[pl.ds(j, L)]
                idst = idst_row[pl.ds(j, L)]
                a = plsc.load_gather(s_v, [isrc])
                b = plsc.load_gather(d_v, [idst])
                m = plsc.load_gather(mp_v, [idst])
                e = a + b
                e = jnp.where(e >= 0.0, e, 0.2 * e)
                ex_row[pl.ds(j, L)] = jnp.exp(e - m)

            pltpu.sync_copy(ex_row.at[pl.ds(0, CH)],
                            den_sh.at[idst_v.at[g]], add=True)

        pltpu.sync_copy(ex_v, ex_hbm.at[wid])
        plsc.subcore_barrier()
        _tile_copyout_1d(den_sh, den_hbm.at[cid], stage1, sid)

    return k(src_r, dst_r, s_n, d_n, mp_n)


def _sc_gat_agg(src_r, dst_r, ex_r, xw2):
    @functools.partial(
        pl.kernel,
        out_type=jax.ShapeDtypeStruct((NC, N, H), jnp.float32),
        mesh=_mesh,
        compiler_params=_cp,
        scratch_types=[
            pltpu.VMEM((NCH, CH), jnp.int32),
            pltpu.VMEM((NCH, CH), jnp.int32),
            pltpu.VMEM((128,), jnp.float32),
            pltpu.VMEM((CH, H), jnp.float32),
            pltpu.VMEM_SHARED((N, H), jnp.float32),
        ],
    )
    def k(src_hbm, dst_hbm, ex_hbm, xw2_hbm, out_hbm,
          isrc_v, idst_v, ex_c, rows_v, acc_sh):
        cid = lax.axis_index("c")
        sid = lax.axis_index("s")
        wid = cid * NS + sid
        pltpu.sync_copy(src_hbm.at[wid], isrc_v)
        pltpu.sync_copy(dst_hbm.at[wid], idst_v)
        _fill_zeros_2d(rows_v)
        _tile_zero_2d(rows_v, acc_sh, sid)
        plsc.subcore_barrier()

        @pl.loop(0, NCH)
        def _(g):
            pltpu.sync_copy(xw2_hbm.at[isrc_v.at[g]], rows_v)
            pltpu.sync_copy(ex_hbm.at[wid, g], ex_c)

            @pl.loop(0, CH)
            def _(i):
                iv = jnp.full((L,), i, jnp.int32)
                w = plsc.load_gather(ex_c, [iv])
                row = rows_v.at[i]
                for kk in range(H // L):
                    row[pl.ds(kk * L, L)] = row[pl.ds(kk * L, L)] * w

            pltpu.sync_copy(rows_v, acc_sh.at[idst_v.at[g]], add=True)

        plsc.subcore_barrier()
        _tile_copyout_2d(acc_sh, out_hbm.at[cid], rows_v, sid)

    return k(src_r, dst_r, ex_r, xw2)


def _sc_readout(src_r, dst_r, p_n, q_n):
    @functools.partial(
        pl.kernel,
        out_type=jax.ShapeDtypeStruct((NW, NCH, 128), jnp.float32),
        mesh=_mesh,
        compiler_params=_cp,
        scratch_types=[
            pltpu.VMEM((NCH, CH), jnp.int32),
            pltpu.VMEM((NCH, CH), jnp.int32),
            pltpu.VMEM((NP,), jnp.float32),
            pltpu.VMEM((NP,), jnp.float32),
            pltpu.VMEM((NCH, 128), jnp.float32),
        ],
    )
    def k(src_hbm, dst_hbm, p_hbm, q_hbm, out_hbm,
          isrc_v, idst_v, p_v, q_v, o_v):
        cid = lax.axis_index("c")
        sid = lax.axis_index("s")
        wid = cid * NS + sid
        pltpu.sync_copy(src_hbm.at[wid], isrc_v)
        pltpu.sync_copy(dst_hbm.at[wid], idst_v)
        pltpu.sync_copy(p_hbm, p_v)
        pltpu.sync_copy(q_hbm, q_v)

        @pl.loop(0, NCH)
        def _(g):
            isrc_row = isrc_v.at[g]
            idst_row = idst_v.at[g]
            o_row = o_v.at[g]

            @pl.loop(0, CH, step=L)
            def _(j):
                isrc = isrc_row[pl.ds(j, L)]
                idst = idst_row[pl.ds(j, L)]
                a = plsc.load_gather(p_v, [isrc])
                b = plsc.load_gather(q_v, [idst])
                o_row[pl.ds(j, L)] = a + b

        pltpu.sync_copy(o_v, out_hbm.at[wid])

    return k(src_r, dst_r, p_n, q_n)


def _tc_xw1(x, W):
    def body(x_ref, w_ref, o_ref):
        o_ref[...] = jnp.dot(x_ref[...], w_ref[...],
                             preferred_element_type=jnp.float32)

    return pl.pallas_call(
        body, out_shape=jax.ShapeDtypeStruct((N, H), jnp.float32))(x, W)


def _tc_y1(xw1, deg_part):
    def body(xw_ref, dp_ref, y_ref, dinv_ref):
        deg = dp_ref[0, :] + dp_ref[1, :] + 1.0
        dinv = lax.rsqrt(deg)
        dinv_ref[...] = dinv[:, None]
        y_ref[...] = xw_ref[...] * dinv[:, None]

    return pl.pallas_call(
        body,
        out_shape=(jax.ShapeDtypeStruct((N, H), jnp.float32),
                   jax.ShapeDtypeStruct((N, 1), jnp.float32)))(xw1, deg_part)


def _tc_gat_prep(acc1_part, y1, dinv, b_gcn, W_gat, a_src, a_dst):
    def body(acc_ref, y_ref, dinv_ref, b_ref, w_ref, asrc_ref, adst_ref,
             xw2_ref, s_ref, d_ref, mp_ref, exs_ref):
        agg = acc_ref[0] + acc_ref[1] + y_ref[...]
        h1 = jnp.maximum(dinv_ref[...] * agg + b_ref[...][None, :], 0.0)
        xw2 = jnp.dot(h1, w_ref[...], preferred_element_type=jnp.float32)
        xw2_ref[...] = xw2
        s = jnp.dot(xw2, asrc_ref[...][:, None],
                    preferred_element_type=jnp.float32)
        d = jnp.dot(xw2, adst_ref[...][:, None],
                    preferred_element_type=jnp.float32)
        s_ref[...] = s
        d_ref[...] = d
        S = jnp.max(s)
        mp = S + d
        mp = jnp.where(mp >= 0.0, mp, 0.2 * mp)
        mp_ref[...] = mp
        es = s + d
        es = jnp.where(es >= 0.0, es, 0.2 * es)
        exs_ref[...] = jnp.exp(es - mp)

    return pl.pallas_call(
        body,
        out_shape=(jax.ShapeDtypeStruct((N, H), jnp.float32),
                   jax.ShapeDtypeStruct((N, 1), jnp.float32),
                   jax.ShapeDtypeStruct((N, 1), jnp.float32),
                   jax.ShapeDtypeStruct((N, 1), jnp.float32),
                   jax.ShapeDtypeStruct((N, 1), jnp.float32)))(
        acc1_part, y1, dinv, b_gcn, W_gat, a_src, a_dst)


def _tc_h2(accD_part, den_part, exs, xw2, b_gat):
    def body(acc_ref, den_ref, exs_ref, xw2_ref, b_ref, h2_ref):
        den = den_ref[0, :] + den_ref[1, :] + exs_ref[...][:, 0]
        rden = 1.0 / (den + 1e-16)
        agg = acc_ref[0] + acc_ref[1] + exs_ref[...] * xw2_ref[...]
        h2_ref[...] = jnp.maximum(rden[:, None] * agg + b_ref[...][None, :],
                                  0.0)

    return pl.pallas_call(
        body, out_shape=jax.ShapeDtypeStruct((N, H), jnp.float32))(
        accD_part, den_part, exs, xw2, b_gat)


def _tc_gin_readout(accE_part, h2, W1, b1, W2, b2, W_fc, b_fc):
    def body(acc_ref, h2_ref, w1_ref, b1_ref, w2_ref, b2_ref, wfc_ref,
             bfc_ref, p_ref, q_ref):
        g = h2_ref[...] + acc_ref[0] + acc_ref[1]
        t = jnp.maximum(
            jnp.dot(g, w1_ref[...], preferred_element_type=jnp.float32)
            + b1_ref[...][None, :], 0.0)
        h3 = jnp.maximum(
            jnp.dot(t, w2_ref[...], preferred_element_type=jnp.float32)
            + b2_ref[...][None, :], 0.0)
        wfc = wfc_ref[...]
        p_ref[...] = (jnp.dot(h3, wfc[:H, :],
                              preferred_element_type=jnp.float32)
                      + bfc_ref[...][None, :])
        q_ref[...] = jnp.dot(h3, wfc[H:, :],
                             preferred_element_type=jnp.float32)

    return pl.pallas_call(
        body,
        out_shape=(jax.ShapeDtypeStruct((N, 1), jnp.float32),
                   jax.ShapeDtypeStruct((N, 1), jnp.float32)))(
        accE_part, h2, W1, b1, W2, b2, W_fc, b_fc)


def _pad_np(v):
    return jnp.pad(v.reshape(N), (0, NP - N))


def kernel(x, edge_index, W_gcn, b_gcn, W_gat, b_gat, a_src, a_dst,
           W1, b1, W2, b2, W_fc, b_fc):
    src_r = edge_index[0].reshape(NW, NCH, CH)
    dst_r = edge_index[1].reshape(NW, NCH, CH)

    deg_part = _sc_degree(dst_r)
    xw1 = _tc_xw1(x, W_gcn)
    y1, dinv = _tc_y1(xw1, deg_part[:, :N])
    acc1_part = _sc_segsum(src_r, dst_r, y1)
    xw2, s, d, mp, exs = _tc_gat_prep(acc1_part, y1, dinv, b_gcn, W_gat,
                                      a_src, a_dst)
    den_part, ex_r = _sc_gat_logits(src_r, dst_r,
                                    _pad_np(s), _pad_np(d), _pad_np(mp))
    accD_part = _sc_gat_agg(src_r, dst_r, ex_r, xw2)
    h2 = _tc_h2(accD_part, den_part[:, :N], exs, xw2, b_gat)
    accE_part = _sc_segsum(src_r, dst_r, h2)
    p, q = _tc_gin_readout(accE_part, h2, W1, b1, W2, b2, W_fc, b_fc)
    out = _sc_readout(src_r, dst_r, _pad_np(p), _pad_np(q))
    return out[:, :, :CH].reshape(E, 1)


# speedup vs baseline: 24.9607x; 1.3834x over previous
import dataclasses
import functools

import jax
import jax.numpy as jnp
from jax import lax
from jax.experimental import pallas as pl
from jax.experimental.pallas import tpu as pltpu
from jax.experimental.pallas import tpu_sc as plsc

N = 10000
E = 320000
H = 128

NC = 2
NS = 16
L = 16
NW = NC * NS
EPW = E // NW
CH = 80
NCH = EPW // CH
NP = 10240

_mesh = plsc.VectorSubcoreMesh(core_axis_name="c", subcore_axis_name="s")

_cp = pltpu.CompilerParams()
if "needs_layout_passes" in pltpu.CompilerParams.__dataclass_fields__:
    _cp = dataclasses.replace(_cp, needs_layout_passes=False)


def _fill_zeros_1d(stage):
    @pl.loop(0, 640, step=L)
    def _(j):
        stage[pl.ds(j, L)] = jnp.zeros((L,), jnp.float32)


def _fill_zeros_2d(stage):
    @pl.loop(0, CH)
    def _(i):
        row = stage.at[i]
        for kk in range(H // L):
            row[pl.ds(kk * L, L)] = jnp.zeros((L,), jnp.float32)


def _tile_zero_1d(stage, sh_ref, sid):
    pltpu.sync_copy(stage, sh_ref.at[pl.ds(sid * 640, 640)])


def _tile_copyout_1d(sh_ref, out_hbm_core, stage, sid):
    pltpu.sync_copy(sh_ref.at[pl.ds(sid * 640, 640)], stage)
    pltpu.sync_copy(stage, out_hbm_core.at[pl.ds(sid * 640, 640)])


def _tile_zero_2d(stage, sh_ref, sid):
    @pl.when(sid < NS - 1)
    def _():
        @pl.loop(0, 8)
        def _(t):
            pltpu.sync_copy(stage, sh_ref.at[pl.ds(sid * 640 + t * CH, CH)])

    @pl.when(sid == NS - 1)
    def _():
        @pl.loop(0, 5)
        def _(t):
            pltpu.sync_copy(stage, sh_ref.at[pl.ds(9600 + t * CH, CH)])


def _tile_copyout_2d(sh_ref, out_hbm_core, stage, sid):
    @pl.when(sid < NS - 1)
    def _():
        @pl.loop(0, 8)
        def _(t):
            pltpu.sync_copy(sh_ref.at[pl.ds(sid * 640 + t * CH, CH)], stage)
            pltpu.sync_copy(stage, out_hbm_core.at[pl.ds(sid * 640 + t * CH,
                                                         CH)])

    @pl.when(sid == NS - 1)
    def _():
        @pl.loop(0, 5)
        def _(t):
            pltpu.sync_copy(sh_ref.at[pl.ds(9600 + t * CH, CH)], stage)
            pltpu.sync_copy(stage, out_hbm_core.at[pl.ds(9600 + t * CH, CH)])


def _sc_degree(dst_r):
    @functools.partial(
        pl.kernel,
        out_type=jax.ShapeDtypeStruct((NC, NP), jnp.float32),
        mesh=_mesh,
        scratch_types=[
            pltpu.VMEM((NCH, CH), jnp.int32),
            pltpu.VMEM((CH,), jnp.float32),
            pltpu.VMEM((640,), jnp.float32),
            pltpu.VMEM_SHARED((NP,), jnp.float32),
        ],
    )
    def k(dst_hbm, out_hbm, idx_v, ones_v, stage1, deg_sh):
        cid = lax.axis_index("c")
        sid = lax.axis_index("s")
        wid = cid * NS + sid
        pltpu.sync_copy(dst_hbm.at[wid], idx_v)

        @pl.loop(0, CH, step=L)
        def _(j):
            ones_v[pl.ds(j, L)] = jnp.full((L,), 1.0, jnp.float32)

        _fill_zeros_1d(stage1)
        _tile_zero_1d(stage1, deg_sh, sid)
        plsc.subcore_barrier()

        @pl.loop(0, NCH)
        def _(g):
            pltpu.sync_copy(ones_v, deg_sh.at[idx_v.at[g]], add=True)

        plsc.subcore_barrier()
        _tile_copyout_1d(deg_sh, out_hbm.at[cid], stage1, sid)

    return k(dst_r)


def _sc_segsum(src_f, dst_r, table):
    @functools.partial(
        pl.kernel,
        out_type=jax.ShapeDtypeStruct((NC, N, H), jnp.float32),
        mesh=_mesh,
        scratch_types=[
            pltpu.VMEM((EPW,), jnp.int32),
            pltpu.VMEM((NCH, CH), jnp.int32),
            pltpu.VMEM((CH, H), jnp.float32),
            pltpu.VMEM((CH, H), jnp.float32),
            pltpu.SemaphoreType.DMA,
            pltpu.SemaphoreType.DMA,
            pltpu.SemaphoreType.DMA,
            pltpu.SemaphoreType.DMA,
            pltpu.VMEM_SHARED((N, H), jnp.float32),
        ],
    )
    def k(src_hbm, dst_hbm, tab_hbm, out_hbm,
          isrc_v, idst_v, rows_a, rows_b, sg_a, sg_b, ss_a, ss_b, acc_sh):
        cid = lax.axis_index("c")
        sid = lax.axis_index("s")
        wid = cid * NS + sid
        pltpu.sync_copy(src_hbm.at[wid], isrc_v)
        pltpu.sync_copy(dst_hbm.at[wid], idst_v)
        _fill_zeros_2d(rows_a)
        _tile_zero_2d(rows_a, acc_sh, sid)
        plsc.subcore_barrier()

        def start_g(g, buf, sem):
            pltpu.async_copy(tab_hbm.at[isrc_v.at[pl.ds(g * CH, CH)]],
                             buf, sem)

        def wait_g(buf, sem):
            pltpu.make_async_copy(tab_hbm.at[isrc_v.at[pl.ds(0, CH)]],
                                  buf, sem).wait()

        def start_s(g, buf, sem):
            pltpu.async_copy(buf, acc_sh.at[idst_v.at[g]], sem, add=True)

        def wait_s(buf, sem):
            pltpu.make_async_copy(buf, acc_sh.at[idst_v.at[0]], sem).wait()

        start_g(0, rows_a, sg_a)

        @pl.loop(0, NCH, step=2)
        def _(g):
            wait_g(rows_a, sg_a)

            @pl.when(g + 1 < NCH)
            def _():
                start_g(g + 1, rows_b, sg_b)

            start_s(g, rows_a, ss_a)

            @pl.when(g + 1 < NCH)
            def _():
                wait_g(rows_b, sg_b)
                wait_s(rows_a, ss_a)

                @pl.when(g + 2 < NCH)
                def _():
                    start_g(g + 2, rows_a, sg_a)

                start_s(g + 1, rows_b, ss_b)
                wait_s(rows_b, ss_b)

            @pl.when(g + 1 >= NCH)
            def _():
                wait_s(rows_a, ss_a)

        plsc.subcore_barrier()
        _tile_copyout_2d(acc_sh, out_hbm.at[cid], rows_a, sid)

    return k(src_f, dst_r, table)


def _sc_gat_logits(src_r, dst_r, s_n, d_n, mp_n):
    @functools.partial(
        pl.kernel,
        out_type=(jax.ShapeDtypeStruct((NC, NP), jnp.float32),
                  jax.ShapeDtypeStruct((NW, NCH, 128), jnp.float32)),
        mesh=_mesh,
        compiler_params=_cp,
        scratch_types=[
            pltpu.VMEM((NCH, CH), jnp.int32),
            pltpu.VMEM((NCH, CH), jnp.int32),
            pltpu.VMEM((NP,), jnp.float32),
            pltpu.VMEM((NP,), jnp.float32),
            pltpu.VMEM((NP,), jnp.float32),
            pltpu.VMEM((NCH, 128), jnp.float32),
            pltpu.VMEM((640,), jnp.float32),
            pltpu.VMEM_SHARED((NP,), jnp.float32),
        ],
    )
    def k(src_hbm, dst_hbm, s_hbm, d_hbm, mp_hbm,
          den_hbm, ex_hbm,
          isrc_v, idst_v, s_v, d_v, mp_v, ex_v, stage1, den_sh):
        cid = lax.axis_index("c")
        sid = lax.axis_index("s")
        wid = cid * NS + sid
        pltpu.sync_copy(src_hbm.at[wid], isrc_v)
        pltpu.sync_copy(dst_hbm.at[wid], idst_v)
        pltpu.sync_copy(s_hbm, s_v)
        pltpu.sync_copy(d_hbm, d_v)
        pltpu.sync_copy(mp_hbm, mp_v)
        _fill_zeros_1d(stage1)
        _tile_zero_1d(stage1, den_sh, sid)
        plsc.subcore_barrier()

        @pl.loop(0, NCH)
        def _(g):
            isrc_row = isrc_v.at[g]
            idst_row = idst_v.at[g]
            ex_row = ex_v.at[g]

            @pl.loop(0, CH, step=L)
            def _(j):
                isrc = isrc_row[pl.ds(j, L)]
                idst = idst_row[pl.ds(j, L)]
                a = plsc.load_gather(s_v, [isrc])
                b = plsc.load_gather(d_v, [idst])
                m = plsc.load_gather(mp_v, [idst])
                e = a + b
                e = jnp.where(e >= 0.0, e, 0.2 * e)
                ex_row[pl.ds(j, L)] = jnp.exp(e - m)

            pltpu.sync_copy(ex_row.at[pl.ds(0, CH)],
                            den_sh.at[idst_v.at[g]], add=True)

        pltpu.sync_copy(ex_v, ex_hbm.at[wid])
        plsc.subcore_barrier()
        _tile_copyout_1d(den_sh, den_hbm.at[cid], stage1, sid)

    return k(src_r, dst_r, s_n, d_n, mp_n)


def _sc_gat_agg(src_f, dst_r, ex_r, xw2):
    @functools.partial(
        pl.kernel,
        out_type=jax.ShapeDtypeStruct((NC, N, H), jnp.float32),
        mesh=_mesh,
        compiler_params=_cp,
        scratch_types=[
            pltpu.VMEM((EPW,), jnp.int32),
            pltpu.VMEM((NCH, CH), jnp.int32),
            pltpu.VMEM((128,), jnp.float32),
            pltpu.VMEM((128,), jnp.float32),
            pltpu.VMEM((CH, H), jnp.float32),
            pltpu.VMEM((CH, H), jnp.float32),
            pltpu.SemaphoreType.DMA,
            pltpu.SemaphoreType.DMA,
            pltpu.SemaphoreType.DMA,
            pltpu.SemaphoreType.DMA,
            pltpu.SemaphoreType.DMA,
            pltpu.SemaphoreType.DMA,
            pltpu.VMEM_SHARED((N, H), jnp.float32),
        ],
    )
    def k(src_hbm, dst_hbm, ex_hbm, xw2_hbm, out_hbm,
          isrc_v, idst_v, ex_a, ex_b, rows_a, rows_b,
          sg_a, sg_b, se_a, se_b, ss_a, ss_b, acc_sh):
        cid = lax.axis_index("c")
        sid = lax.axis_index("s")
        wid = cid * NS + sid
        pltpu.sync_copy(src_hbm.at[wid], isrc_v)
        pltpu.sync_copy(dst_hbm.at[wid], idst_v)
        _fill_zeros_2d(rows_a)
        _tile_zero_2d(rows_a, acc_sh, sid)
        plsc.subcore_barrier()

        def start_g(g, buf, exbuf, semg, seme):
            pltpu.async_copy(xw2_hbm.at[isrc_v.at[pl.ds(g * CH, CH)]],
                             buf, semg)
            pltpu.async_copy(ex_hbm.at[wid, g], exbuf, seme)

        def wait_g(buf, exbuf, semg, seme):
            pltpu.make_async_copy(xw2_hbm.at[isrc_v.at[pl.ds(0, CH)]],
                                  buf, semg).wait()
            pltpu.make_async_copy(ex_hbm.at[wid, 0], exbuf, seme).wait()

        def start_s(g, buf, sem):
            pltpu.async_copy(buf, acc_sh.at[idst_v.at[g]], sem, add=True)

        def wait_s(buf, sem):
            pltpu.make_async_copy(buf, acc_sh.at[idst_v.at[0]], sem).wait()

        def scale(buf, exbuf):
            @pl.loop(0, CH, unroll=4)
            def _(i):
                iv = jnp.full((L,), i, jnp.int32)
                w = plsc.load_gather(exbuf, [iv])
                row = buf.at[i]
                for kk in range(H // L):
                    row[pl.ds(kk * L, L)] = row[pl.ds(kk * L, L)] * w

        start_g(0, rows_a, ex_a, sg_a, se_a)

        @pl.loop(0, NCH, step=2)
        def _(g):
            wait_g(rows_a, ex_a, sg_a, se_a)

            @pl.when(g + 1 < NCH)
            def _():
                start_g(g + 1, rows_b, ex_b, sg_b, se_b)

            scale(rows_a, ex_a)
            start_s(g, rows_a, ss_a)

            @pl.when(g + 1 < NCH)
            def _():
                wait_g(rows_b, ex_b, sg_b, se_b)
                scale(rows_b, ex_b)
                wait_s(rows_a, ss_a)

                @pl.when(g + 2 < NCH)
                def _():
                    start_g(g + 2, rows_a, ex_a, sg_a, se_a)

                start_s(g + 1, rows_b, ss_b)
                wait_s(rows_b, ss_b)

            @pl.when(g + 1 >= NCH)
            def _():
                wait_s(rows_a, ss_a)

        plsc.subcore_barrier()
        _tile_copyout_2d(acc_sh, out_hbm.at[cid], rows_a, sid)

    return k(src_f, dst_r, ex_r, xw2)


def _sc_readout(src_r, dst_r, p_n, q_n):
    @functools.partial(
        pl.kernel,
        out_type=jax.ShapeDtypeStruct((NW, NCH, 128), jnp.float32),
        mesh=_mesh,
        compiler_params=_cp,
        scratch_types=[
            pltpu.VMEM((NCH, CH), jnp.int32),
            pltpu.VMEM((NCH, CH), jnp.int32),
            pltpu.VMEM((NP,), jnp.float32),
            pltpu.VMEM((NP,), jnp.float32),
            pltpu.VMEM((NCH, 128), jnp.float32),
        ],
    )
    def k(src_hbm, dst_hbm, p_hbm, q_hbm, out_hbm,
          isrc_v, idst_v, p_v, q_v, o_v):
        cid = lax.axis_index("c")
        sid = lax.axis_index("s")
        wid = cid * NS + sid
        pltpu.sync_copy(src_hbm.at[wid], isrc_v)
        pltpu.sync_copy(dst_hbm.at[wid], idst_v)
        pltpu.sync_copy(p_hbm, p_v)
        pltpu.sync_copy(q_hbm, q_v)

        @pl.loop(0, NCH)
        def _(g):
            isrc_row = isrc_v.at[g]
            idst_row = idst_v.at[g]
            o_row = o_v.at[g]

            @pl.loop(0, CH, step=L)
            def _(j):
                isrc = isrc_row[pl.ds(j, L)]
                idst = idst_row[pl.ds(j, L)]
                a = plsc.load_gather(p_v, [isrc])
                b = plsc.load_gather(q_v, [idst])
                o_row[pl.ds(j, L)] = a + b

        pltpu.sync_copy(o_v, out_hbm.at[wid])

    return k(src_r, dst_r, p_n, q_n)


def _tc_xw1(x, W):
    def body(x_ref, w_ref, o_ref):
        o_ref[...] = jnp.dot(x_ref[...], w_ref[...],
                             preferred_element_type=jnp.float32)

    return pl.pallas_call(
        body, out_shape=jax.ShapeDtypeStruct((N, H), jnp.float32))(x, W)


def _tc_y1(xw1, deg_part):
    def body(xw_ref, dp_ref, y_ref, dinv_ref):
        deg = dp_ref[0, :] + dp_ref[1, :] + 1.0
        dinv = lax.rsqrt(deg)
        dinv_ref[...] = dinv[:, None]
        y_ref[...] = xw_ref[...] * dinv[:, None]

    return pl.pallas_call(
        body,
        out_shape=(jax.ShapeDtypeStruct((N, H), jnp.float32),
                   jax.ShapeDtypeStruct((N, 1), jnp.float32)))(xw1, deg_part)


def _tc_gat_prep(acc1_part, y1, dinv, b_gcn, W_gat, a_src, a_dst):
    def body(acc_ref, y_ref, dinv_ref, b_ref, w_ref, asrc_ref, adst_ref,
             xw2_ref, s_ref, d_ref, mp_ref, exs_ref):
        agg = acc_ref[0] + acc_ref[1] + y_ref[...]
        h1 = jnp.maximum(dinv_ref[...] * agg + b_ref[...][None, :], 0.0)
        xw2 = jnp.dot(h1, w_ref[...], preferred_element_type=jnp.float32)
        xw2_ref[...] = xw2
        s = jnp.dot(xw2, asrc_ref[...][:, None],
                    preferred_element_type=jnp.float32)
        d = jnp.dot(xw2, adst_ref[...][:, None],
                    preferred_element_type=jnp.float32)
        s_ref[...] = s
        d_ref[...] = d
        S = jnp.max(s)
        mp = S + d
        mp = jnp.where(mp >= 0.0, mp, 0.2 * mp)
        mp_ref[...] = mp
        es = s + d
        es = jnp.where(es >= 0.0, es, 0.2 * es)
        exs_ref[...] = jnp.exp(es - mp)

    return pl.pallas_call(
        body,
        out_shape=(jax.ShapeDtypeStruct((N, H), jnp.float32),
                   jax.ShapeDtypeStruct((N, 1), jnp.float32),
                   jax.ShapeDtypeStruct((N, 1), jnp.float32),
                   jax.ShapeDtypeStruct((N, 1), jnp.float32),
                   jax.ShapeDtypeStruct((N, 1), jnp.float32)))(
        acc1_part, y1, dinv, b_gcn, W_gat, a_src, a_dst)


def _tc_h2(accD_part, den_part, exs, xw2, b_gat):
    def body(acc_ref, den_ref, exs_ref, xw2_ref, b_ref, h2_ref):
        den = den_ref[0, :] + den_ref[1, :] + exs_ref[...][:, 0]
        rden = 1.0 / (den + 1e-16)
        agg = acc_ref[0] + acc_ref[1] + exs_ref[...] * xw2_ref[...]
        h2_ref[...] = jnp.maximum(rden[:, None] * agg + b_ref[...][None, :],
                                  0.0)

    return pl.pallas_call(
        body, out_shape=jax.ShapeDtypeStruct((N, H), jnp.float32))(
        accD_part, den_part, exs, xw2, b_gat)


def _tc_gin_readout(accE_part, h2, W1, b1, W2, b2, W_fc, b_fc):
    def body(acc_ref, h2_ref, w1_ref, b1_ref, w2_ref, b2_ref, wfc_ref,
             bfc_ref, p_ref, q_ref):
        g = h2_ref[...] + acc_ref[0] + acc_ref[1]
        t = jnp.maximum(
            jnp.dot(g, w1_ref[...], preferred_element_type=jnp.float32)
            + b1_ref[...][None, :], 0.0)
        h3 = jnp.maximum(
            jnp.dot(t, w2_ref[...], preferred_element_type=jnp.float32)
            + b2_ref[...][None, :], 0.0)
        wfc = wfc_ref[...]
        p_ref[...] = (jnp.dot(h3, wfc[:H, :],
                              preferred_element_type=jnp.float32)
                      + bfc_ref[...][None, :])
        q_ref[...] = jnp.dot(h3, wfc[H:, :],
                             preferred_element_type=jnp.float32)

    return pl.pallas_call(
        body,
        out_shape=(jax.ShapeDtypeStruct((N, 1), jnp.float32),
                   jax.ShapeDtypeStruct((N, 1), jnp.float32)))(
        accE_part, h2, W1, b1, W2, b2, W_fc, b_fc)


def _pad_np(v):
    return jnp.pad(v.reshape(N), (0, NP - N))


def kernel(x, edge_index, W_gcn, b_gcn, W_gat, b_gat, a_src, a_dst,
           W1, b1, W2, b2, W_fc, b_fc):
    src_r = edge_index[0].reshape(NW, NCH, CH)
    src_f = edge_index[0].reshape(NW, EPW)
    dst_r = edge_index[1].reshape(NW, NCH, CH)

    deg_part = _sc_degree(dst_r)
    xw1 = _tc_xw1(x, W_gcn)
    y1, dinv = _tc_y1(xw1, deg_part[:, :N])
    acc1_part = _sc_segsum(src_f, dst_r, y1)
    xw2, s, d, mp, exs = _tc_gat_prep(acc1_part, y1, dinv, b_gcn, W_gat,
                                      a_src, a_dst)
    den_part, ex_r = _sc_gat_logits(src_r, dst_r,
                                    _pad_np(s), _pad_np(d), _pad_np(mp))
    accD_part = _sc_gat_agg(src_f, dst_r, ex_r, xw2)
    h2 = _tc_h2(accD_part, den_part[:, :N], exs, xw2, b_gat)
    accE_part = _sc_segsum(src_f, dst_r, h2)
    p, q = _tc_gin_readout(accE_part, h2, W1, b1, W2, b2, W_fc, b_fc)
    out = _sc_readout(src_r, dst_r, _pad_np(p), _pad_np(q))
    return out[:, :, :CH].reshape(E, 1)


# speedup vs baseline: 26.2804x; 1.0529x over previous
import dataclasses
import functools

import jax
import jax.numpy as jnp
from jax import lax
from jax.experimental import pallas as pl
from jax.experimental.pallas import tpu as pltpu
from jax.experimental.pallas import tpu_sc as plsc

N = 10000
E = 320000
H = 128

NC = 2
NS = 16
L = 16
NW = NC * NS
EPW = E // NW
CH = 80
NCH = EPW // CH
NP = 10240

_mesh = plsc.VectorSubcoreMesh(core_axis_name="c", subcore_axis_name="s")

_cp = pltpu.CompilerParams()
if "needs_layout_passes" in pltpu.CompilerParams.__dataclass_fields__:
    _cp = dataclasses.replace(_cp, needs_layout_passes=False)


def _fill_zeros_1d(stage):
    @pl.loop(0, 640, step=L)
    def _(j):
        stage[pl.ds(j, L)] = jnp.zeros((L,), jnp.float32)


def _fill_zeros_2d(stage):
    @pl.loop(0, CH)
    def _(i):
        row = stage.at[i]
        for kk in range(H // L):
            row[pl.ds(kk * L, L)] = jnp.zeros((L,), jnp.float32)


def _tile_zero_1d(stage, sh_ref, sid):
    pltpu.sync_copy(stage, sh_ref.at[pl.ds(sid * 640, 640)])


def _tile_copyout_1d(sh_ref, out_hbm_core, stage, sid):
    pltpu.sync_copy(sh_ref.at[pl.ds(sid * 640, 640)], stage)
    pltpu.sync_copy(stage, out_hbm_core.at[pl.ds(sid * 640, 640)])


def _tile_zero_2d(stage, sh_ref, sid):
    @pl.when(sid < NS - 1)
    def _():
        @pl.loop(0, 8)
        def _(t):
            pltpu.sync_copy(stage, sh_ref.at[pl.ds(sid * 640 + t * CH, CH)])

    @pl.when(sid == NS - 1)
    def _():
        @pl.loop(0, 5)
        def _(t):
            pltpu.sync_copy(stage, sh_ref.at[pl.ds(9600 + t * CH, CH)])


def _tile_copyout_2d(sh_ref, out_hbm_core, stage, sid):
    @pl.when(sid < NS - 1)
    def _():
        @pl.loop(0, 8)
        def _(t):
            pltpu.sync_copy(sh_ref.at[pl.ds(sid * 640 + t * CH, CH)], stage)
            pltpu.sync_copy(stage, out_hbm_core.at[pl.ds(sid * 640 + t * CH,
                                                         CH)])

    @pl.when(sid == NS - 1)
    def _():
        @pl.loop(0, 5)
        def _(t):
            pltpu.sync_copy(sh_ref.at[pl.ds(9600 + t * CH, CH)], stage)
            pltpu.sync_copy(stage, out_hbm_core.at[pl.ds(9600 + t * CH, CH)])


def _sc_degree(dst_r):
    @functools.partial(
        pl.kernel,
        out_type=jax.ShapeDtypeStruct((NC, NP), jnp.float32),
        mesh=_mesh,
        scratch_types=[
            pltpu.VMEM((NCH, CH), jnp.int32),
            pltpu.VMEM((CH,), jnp.float32),
            pltpu.VMEM((640,), jnp.float32),
            pltpu.VMEM_SHARED((NP,), jnp.float32),
        ],
    )
    def k(dst_hbm, out_hbm, idx_v, ones_v, stage1, deg_sh):
        cid = lax.axis_index("c")
        sid = lax.axis_index("s")
        wid = cid * NS + sid
        pltpu.sync_copy(dst_hbm.at[wid], idx_v)

        @pl.loop(0, CH, step=L)
        def _(j):
            ones_v[pl.ds(j, L)] = jnp.full((L,), 1.0, jnp.float32)

        _fill_zeros_1d(stage1)
        _tile_zero_1d(stage1, deg_sh, sid)
        plsc.subcore_barrier()

        @pl.loop(0, NCH)
        def _(g):
            pltpu.sync_copy(ones_v, deg_sh.at[idx_v.at[g]], add=True)

        plsc.subcore_barrier()
        _tile_copyout_1d(deg_sh, out_hbm.at[cid], stage1, sid)

    return k(dst_r)


def _sc_segsum(src_f, dst_r, table):
    @functools.partial(
        pl.kernel,
        out_type=jax.ShapeDtypeStruct((NC, N, H), jnp.float32),
        mesh=_mesh,
        scratch_types=[
            pltpu.VMEM((EPW,), jnp.int32),
            pltpu.VMEM((NCH, CH), jnp.int32),
            pltpu.VMEM((CH, H), jnp.float32),
            pltpu.VMEM((CH, H), jnp.float32),
            (pltpu.SemaphoreType.DMA, pltpu.SemaphoreType.DMA),
            (pltpu.SemaphoreType.DMA, pltpu.SemaphoreType.DMA),
            pltpu.SemaphoreType.DMA,
            pltpu.SemaphoreType.DMA,
            pltpu.VMEM_SHARED((N, H), jnp.float32),
        ],
    )
    def k(src_hbm, dst_hbm, tab_hbm, out_hbm,
          isrc_v, idst_v, rows_a, rows_b, sg_a, sg_b, ss_a, ss_b, acc_sh):
        cid = lax.axis_index("c")
        sid = lax.axis_index("s")
        wid = cid * NS + sid
        pltpu.sync_copy(src_hbm.at[wid], isrc_v)
        pltpu.sync_copy(dst_hbm.at[wid], idst_v)
        _fill_zeros_2d(rows_a)
        _tile_zero_2d(rows_a, acc_sh, sid)
        plsc.subcore_barrier()

        HC = CH // 2

        def start_g(g, buf, sem):
            pltpu.async_copy(tab_hbm.at[isrc_v.at[pl.ds(g * CH, HC)]],
                             buf.at[pl.ds(0, HC)], sem[0])
            pltpu.async_copy(tab_hbm.at[isrc_v.at[pl.ds(g * CH + HC, HC)]],
                             buf.at[pl.ds(HC, HC)], sem[1])

        def wait_g(buf, sem):
            pltpu.make_async_copy(tab_hbm.at[isrc_v.at[pl.ds(0, HC)]],
                                  buf.at[pl.ds(0, HC)], sem[0]).wait()
            pltpu.make_async_copy(tab_hbm.at[isrc_v.at[pl.ds(0, HC)]],
                                  buf.at[pl.ds(HC, HC)], sem[1]).wait()

        def start_s(g, buf, sem):
            pltpu.async_copy(buf, acc_sh.at[idst_v.at[g]], sem, add=True)

        def wait_s(buf, sem):
            pltpu.make_async_copy(buf, acc_sh.at[idst_v.at[0]], sem).wait()

        start_g(0, rows_a, sg_a)

        @pl.loop(0, NCH, step=2)
        def _(g):
            wait_g(rows_a, sg_a)

            @pl.when(g + 1 < NCH)
            def _():
                start_g(g + 1, rows_b, sg_b)

            start_s(g, rows_a, ss_a)

            @pl.when(g + 1 < NCH)
            def _():
                wait_g(rows_b, sg_b)
                wait_s(rows_a, ss_a)

                @pl.when(g + 2 < NCH)
                def _():
                    start_g(g + 2, rows_a, sg_a)

                start_s(g + 1, rows_b, ss_b)
                wait_s(rows_b, ss_b)

            @pl.when(g + 1 >= NCH)
            def _():
                wait_s(rows_a, ss_a)

        plsc.subcore_barrier()
        _tile_copyout_2d(acc_sh, out_hbm.at[cid], rows_a, sid)

    return k(src_f, dst_r, table)


def _sc_gat_logits(src_r, dst_r, s_n, d_n, mp_n):
    @functools.partial(
        pl.kernel,
        out_type=(jax.ShapeDtypeStruct((NC, NP), jnp.float32),
                  jax.ShapeDtypeStruct((NW, NCH, 128), jnp.float32)),
        mesh=_mesh,
        compiler_params=_cp,
        scratch_types=[
            pltpu.VMEM((NCH, CH), jnp.int32),
            pltpu.VMEM((NCH, CH), jnp.int32),
            pltpu.VMEM((NP,), jnp.float32),
            pltpu.VMEM((NP,), jnp.float32),
            pltpu.VMEM((NP,), jnp.float32),
            pltpu.VMEM((NCH, 128), jnp.float32),
            pltpu.VMEM((640,), jnp.float32),
            pltpu.SemaphoreType.DMA,
            pltpu.SemaphoreType.DMA,
            pltpu.SemaphoreType.DMA,
            pltpu.SemaphoreType.DMA,
            pltpu.VMEM_SHARED((NP,), jnp.float32),
        ],
    )
    def k(src_hbm, dst_hbm, s_hbm, d_hbm, mp_hbm,
          den_hbm, ex_hbm,
          isrc_v, idst_v, s_v, d_v, mp_v, ex_v, stage1,
          sem_a, sem_b, sem_c, sem_d, den_sh):
        cid = lax.axis_index("c")
        sid = lax.axis_index("s")
        wid = cid * NS + sid
        pltpu.async_copy(src_hbm.at[wid], isrc_v, sem_a)
        pltpu.async_copy(dst_hbm.at[wid], idst_v, sem_b)
        pltpu.async_copy(s_hbm, s_v, sem_c)
        pltpu.async_copy(d_hbm, d_v, sem_d)
        pltpu.sync_copy(mp_hbm, mp_v)
        pltpu.make_async_copy(src_hbm.at[wid], isrc_v, sem_a).wait()
        pltpu.make_async_copy(dst_hbm.at[wid], idst_v, sem_b).wait()
        pltpu.make_async_copy(s_hbm, s_v, sem_c).wait()
        pltpu.make_async_copy(d_hbm, d_v, sem_d).wait()
        _fill_zeros_1d(stage1)
        _tile_zero_1d(stage1, den_sh, sid)
        plsc.subcore_barrier()

        @pl.loop(0, NCH)
        def _(g):
            isrc_row = isrc_v.at[g]
            idst_row = idst_v.at[g]
            ex_row = ex_v.at[g]

            @pl.loop(0, CH, step=L)
            def _(j):
                isrc = isrc_row[pl.ds(j, L)]
                idst = idst_row[pl.ds(j, L)]
                a = plsc.load_gather(s_v, [isrc])
                b = plsc.load_gather(d_v, [idst])
                m = plsc.load_gather(mp_v, [idst])
                e = a + b
                e = jnp.where(e >= 0.0, e, 0.2 * e)
                ex_row[pl.ds(j, L)] = jnp.exp(e - m)

            pltpu.sync_copy(ex_row.at[pl.ds(0, CH)],
                            den_sh.at[idst_v.at[g]], add=True)

        pltpu.sync_copy(ex_v, ex_hbm.at[wid])
        plsc.subcore_barrier()
        _tile_copyout_1d(den_sh, den_hbm.at[cid], stage1, sid)

    return k(src_r, dst_r, s_n, d_n, mp_n)


def _sc_gat_agg(src_f, dst_r, ex_r, xw2):
    @functools.partial(
        pl.kernel,
        out_type=jax.ShapeDtypeStruct((NC, N, H), jnp.float32),
        mesh=_mesh,
        compiler_params=_cp,
        scratch_types=[
            pltpu.VMEM((EPW,), jnp.int32),
            pltpu.VMEM((NCH, CH), jnp.int32),
            pltpu.VMEM((128,), jnp.float32),
            pltpu.VMEM((128,), jnp.float32),
            pltpu.VMEM((CH, H), jnp.float32),
            pltpu.VMEM((CH, H), jnp.float32),
            (pltpu.SemaphoreType.DMA, pltpu.SemaphoreType.DMA),
            (pltpu.SemaphoreType.DMA, pltpu.SemaphoreType.DMA),
            pltpu.SemaphoreType.DMA,
            pltpu.SemaphoreType.DMA,
            pltpu.SemaphoreType.DMA,
            pltpu.SemaphoreType.DMA,
            pltpu.VMEM_SHARED((N, H), jnp.float32),
        ],
    )
    def k(src_hbm, dst_hbm, ex_hbm, xw2_hbm, out_hbm,
          isrc_v, idst_v, ex_a, ex_b, rows_a, rows_b,
          sg_a, sg_b, se_a, se_b, ss_a, ss_b, acc_sh):
        cid = lax.axis_index("c")
        sid = lax.axis_index("s")
        wid = cid * NS + sid
        pltpu.sync_copy(src_hbm.at[wid], isrc_v)
        pltpu.sync_copy(dst_hbm.at[wid], idst_v)
        _fill_zeros_2d(rows_a)
        _tile_zero_2d(rows_a, acc_sh, sid)
        plsc.subcore_barrier()

        HC = CH // 2

        def start_g(g, buf, exbuf, semg, seme):
            pltpu.async_copy(xw2_hbm.at[isrc_v.at[pl.ds(g * CH, HC)]],
                             buf.at[pl.ds(0, HC)], semg[0])
            pltpu.async_copy(xw2_hbm.at[isrc_v.at[pl.ds(g * CH + HC, HC)]],
                             buf.at[pl.ds(HC, HC)], semg[1])
            pltpu.async_copy(ex_hbm.at[wid, g], exbuf, seme)

        def wait_g(buf, exbuf, semg, seme):
            pltpu.make_async_copy(xw2_hbm.at[isrc_v.at[pl.ds(0, HC)]],
                                  buf.at[pl.ds(0, HC)], semg[0]).wait()
            pltpu.make_async_copy(xw2_hbm.at[isrc_v.at[pl.ds(0, HC)]],
                                  buf.at[pl.ds(HC, HC)], semg[1]).wait()
            pltpu.make_async_copy(ex_hbm.at[wid, 0], exbuf, seme).wait()

        def start_s(g, buf, sem):
            pltpu.async_copy(buf, acc_sh.at[idst_v.at[g]], sem, add=True)

        def wait_s(buf, sem):
            pltpu.make_async_copy(buf, acc_sh.at[idst_v.at[0]], sem).wait()

        def scale(buf, exbuf):
            @pl.loop(0, CH, unroll=8)
            def _(i):
                iv = jnp.full((L,), i, jnp.int32)
                w = plsc.load_gather(exbuf, [iv])
                row = buf.at[i]
                for kk in range(H // L):
                    row[pl.ds(kk * L, L)] = row[pl.ds(kk * L, L)] * w

        start_g(0, rows_a, ex_a, sg_a, se_a)

        @pl.loop(0, NCH, step=2)
        def _(g):
            wait_g(rows_a, ex_a, sg_a, se_a)

            @pl.when(g + 1 < NCH)
            def _():
                start_g(g + 1, rows_b, ex_b, sg_b, se_b)

            scale(rows_a, ex_a)
            start_s(g, rows_a, ss_a)

            @pl.when(g + 1 < NCH)
            def _():
                wait_g(rows_b, ex_b, sg_b, se_b)
                scale(rows_b, ex_b)
                wait_s(rows_a, ss_a)

                @pl.when(g + 2 < NCH)
                def _():
                    start_g(g + 2, rows_a, ex_a, sg_a, se_a)

                start_s(g + 1, rows_b, ss_b)
                wait_s(rows_b, ss_b)

            @pl.when(g + 1 >= NCH)
            def _():
                wait_s(rows_a, ss_a)

        plsc.subcore_barrier()
        _tile_copyout_2d(acc_sh, out_hbm.at[cid], rows_a, sid)

    return k(src_f, dst_r, ex_r, xw2)


def _sc_readout(src_r, dst_r, p_n, q_n):
    @functools.partial(
        pl.kernel,
        out_type=jax.ShapeDtypeStruct((NW, NCH, 128), jnp.float32),
        mesh=_mesh,
        compiler_params=_cp,
        scratch_types=[
            pltpu.VMEM((NCH, CH), jnp.int32),
            pltpu.VMEM((NCH, CH), jnp.int32),
            pltpu.VMEM((NP,), jnp.float32),
            pltpu.VMEM((NP,), jnp.float32),
            pltpu.VMEM((NCH, 128), jnp.float32),
        ],
    )
    def k(src_hbm, dst_hbm, p_hbm, q_hbm, out_hbm,
          isrc_v, idst_v, p_v, q_v, o_v):
        cid = lax.axis_index("c")
        sid = lax.axis_index("s")
        wid = cid * NS + sid
        pltpu.sync_copy(src_hbm.at[wid], isrc_v)
        pltpu.sync_copy(dst_hbm.at[wid], idst_v)
        pltpu.sync_copy(p_hbm, p_v)
        pltpu.sync_copy(q_hbm, q_v)

        @pl.loop(0, NCH)
        def _(g):
            isrc_row = isrc_v.at[g]
            idst_row = idst_v.at[g]
            o_row = o_v.at[g]

            @pl.loop(0, CH, step=L)
            def _(j):
                isrc = isrc_row[pl.ds(j, L)]
                idst = idst_row[pl.ds(j, L)]
                a = plsc.load_gather(p_v, [isrc])
                b = plsc.load_gather(q_v, [idst])
                o_row[pl.ds(j, L)] = a + b

        pltpu.sync_copy(o_v, out_hbm.at[wid])

    return k(src_r, dst_r, p_n, q_n)


def _tc_y1(x, W, deg_part):
    def body(x_ref, w_ref, dp_ref, y_ref, dinv_ref):
        xw = jnp.dot(x_ref[...], w_ref[...],
                     preferred_element_type=jnp.float32)
        deg = dp_ref[0, :] + dp_ref[1, :] + 1.0
        dinv = lax.rsqrt(deg)
        dinv_ref[...] = dinv[:, None]
        y_ref[...] = xw * dinv[:, None]

    return pl.pallas_call(
        body,
        out_shape=(jax.ShapeDtypeStruct((N, H), jnp.float32),
                   jax.ShapeDtypeStruct((N, 1), jnp.float32)))(x, W, deg_part)


def _tc_gat_prep(acc1_part, y1, dinv, b_gcn, W_gat, a_src, a_dst):
    def body(acc_ref, y_ref, dinv_ref, b_ref, w_ref, asrc_ref, adst_ref,
             xw2_ref, s_ref, d_ref, mp_ref, exs_ref):
        agg = acc_ref[0] + acc_ref[1] + y_ref[...]
        h1 = jnp.maximum(dinv_ref[...] * agg + b_ref[...][None, :], 0.0)
        xw2 = jnp.dot(h1, w_ref[...], preferred_element_type=jnp.float32)
        xw2_ref[...] = xw2
        s = jnp.dot(xw2, asrc_ref[...][:, None],
                    preferred_element_type=jnp.float32)
        d = jnp.dot(xw2, adst_ref[...][:, None],
                    preferred_element_type=jnp.float32)
        s_ref[...] = s
        d_ref[...] = d
        S = jnp.max(s)
        mp = S + d
        mp = jnp.where(mp >= 0.0, mp, 0.2 * mp)
        mp_ref[...] = mp
        es = s + d
        es = jnp.where(es >= 0.0, es, 0.2 * es)
        exs_ref[...] = jnp.exp(es - mp)

    return pl.pallas_call(
        body,
        out_shape=(jax.ShapeDtypeStruct((N, H), jnp.float32),
                   jax.ShapeDtypeStruct((N, 1), jnp.float32),
                   jax.ShapeDtypeStruct((N, 1), jnp.float32),
                   jax.ShapeDtypeStruct((N, 1), jnp.float32),
                   jax.ShapeDtypeStruct((N, 1), jnp.float32)))(
        acc1_part, y1, dinv, b_gcn, W_gat, a_src, a_dst)


def _tc_h2(accD_part, den_part, exs, xw2, b_gat):
    def body(acc_ref, den_ref, exs_ref, xw2_ref, b_ref, h2_ref):
        den = den_ref[0, :] + den_ref[1, :] + exs_ref[...][:, 0]
        rden = 1.0 / (den + 1e-16)
        agg = acc_ref[0] + acc_ref[1] + exs_ref[...] * xw2_ref[...]
        h2_ref[...] = jnp.maximum(rden[:, None] * agg + b_ref[...][None, :],
                                  0.0)

    return pl.pallas_call(
        body, out_shape=jax.ShapeDtypeStruct((N, H), jnp.float32))(
        accD_part, den_part, exs, xw2, b_gat)


def _tc_gin_readout(accE_part, h2, W1, b1, W2, b2, W_fc, b_fc):
    def body(acc_ref, h2_ref, w1_ref, b1_ref, w2_ref, b2_ref, wfc_ref,
             bfc_ref, p_ref, q_ref):
        g = h2_ref[...] + acc_ref[0] + acc_ref[1]
        t = jnp.maximum(
            jnp.dot(g, w1_ref[...], preferred_element_type=jnp.float32)
            + b1_ref[...][None, :], 0.0)
        h3 = jnp.maximum(
            jnp.dot(t, w2_ref[...], preferred_element_type=jnp.float32)
            + b2_ref[...][None, :], 0.0)
        wfc = wfc_ref[...]
        p_ref[...] = (jnp.dot(h3, wfc[:H, :],
                              preferred_element_type=jnp.float32)
                      + bfc_ref[...][None, :])
        q_ref[...] = jnp.dot(h3, wfc[H:, :],
                             preferred_element_type=jnp.float32)

    return pl.pallas_call(
        body,
        out_shape=(jax.ShapeDtypeStruct((N, 1), jnp.float32),
                   jax.ShapeDtypeStruct((N, 1), jnp.float32)))(
        accE_part, h2, W1, b1, W2, b2, W_fc, b_fc)


def _pad_np(v):
    return jnp.pad(v.reshape(N), (0, NP - N))


def kernel(x, edge_index, W_gcn, b_gcn, W_gat, b_gat, a_src, a_dst,
           W1, b1, W2, b2, W_fc, b_fc):
    src_r = edge_index[0].reshape(NW, NCH, CH)
    src_f = edge_index[0].reshape(NW, EPW)
    dst_r = edge_index[1].reshape(NW, NCH, CH)

    deg_part = _sc_degree(dst_r)
    y1, dinv = _tc_y1(x, W_gcn, deg_part[:, :N])
    acc1_part = _sc_segsum(src_f, dst_r, y1)
    xw2, s, d, mp, exs = _tc_gat_prep(acc1_part, y1, dinv, b_gcn, W_gat,
                                      a_src, a_dst)
    den_part, ex_r = _sc_gat_logits(src_r, dst_r,
                                    _pad_np(s), _pad_np(d), _pad_np(mp))
    accD_part = _sc_gat_agg(src_f, dst_r, ex_r, xw2)
    h2 = _tc_h2(accD_part, den_part[:, :N], exs, xw2, b_gat)
    accE_part = _sc_segsum(src_f, dst_r, h2)
    p, q = _tc_gin_readout(accE_part, h2, W1, b1, W2, b2, W_fc, b_fc)
    out = _sc_readout(src_r, dst_r, _pad_np(p), _pad_np(q))
    return out[:, :, :CH].reshape(E, 1)
